# baseline (speedup 1.0000x reference)
import jax
import jax.numpy as jnp
from jax import lax
from jax.experimental import pallas as pl
from jax.experimental.pallas import tpu as pltpu
from jax.experimental.pallas import tpu_sc as plsc

V = 100000
D = 128
B = 4096
L = 200
LANES = 16
NG = D // 32

CH0 = 128
CH1 = L - CH0
NBUF = 4

NC = 2
NS = 16
NW = NC * NS
BPW = B // NW
GROUPS = BPW // 2

RPW = V // NW
RPC = 125
NCHC = RPW // RPC

MASK_HI = jnp.int32(-65536)
ROUND = jnp.int32(0x8000)

_PARAMS = pltpu.CompilerParams(use_tc_tiling_on_sc=False,
                               needs_layout_passes=False)
_MESH = dict(core_axis_name="c", subcore_axis_name="s")


def _wid():
    return lax.axis_index("s") * NC + lax.axis_index("c")


def _cvt_body(table_hbm, tb_hbm, in_v, out_v, semi0, semi1, semo0, semo1):
    semi = (semi0, semi1)
    semo = (semo0, semo1)
    base = _wid() * RPW

    def start_in(c, u):
        pltpu.async_copy(table_hbm.at[pl.ds(base + c * RPC, RPC)],
                         in_v.at[u], semi[u])

    def wait_in(u):
        pltpu.make_async_copy(table_hbm.at[pl.ds(0, RPC)], in_v.at[u],
                              semi[u]).wait()

    def start_out(c, u):
        pltpu.async_copy(out_v.at[u], tb_hbm.at[pl.ds(base + c * RPC, RPC)],
                         semo[u])

    def wait_out(u):
        pltpu.make_async_copy(out_v.at[u], tb_hbm.at[pl.ds(0, RPC)],
                              semo[u]).wait()

    start_in(0, 0)
    start_in(1, 1)
    for c in range(NCHC):
        u = c % 2
        wait_in(u)
        if c >= 2:
            wait_out(u)

        def rbody(j, carry, u=u):
            for r5 in range(5):
                r = 5 * j + r5
                row_in = in_v.at[u, r]
                for m in range(NG):
                    a = plsc.bitcast(row_in[pl.ds(32 * m, LANES)], jnp.int32)
                    b = plsc.bitcast(row_in[pl.ds(32 * m + 16, LANES)],
                                     jnp.int32)
                    lo = lax.shift_right_logical(a + ROUND, 16)
                    hi = (b + ROUND) & MASK_HI
                    out_v[u, r, pl.ds(32 * m, 32)] = plsc.bitcast(
                        lo | hi, jnp.bfloat16)
            return carry

        lax.fori_loop(0, RPC // 5, rbody, 0)
        start_out(c, u)
        if c + 2 < NCHC:
            start_in(c + 2, u)
    wait_out(NCHC % 2)
    wait_out((NCHC + 1) % 2)


def _chunk(u):
    return u // 2, (u % 2) * CH0, CH0 if u % 2 == 0 else CH1


def _sc_body(ids_hbm, table_hbm, out_hbm, idx_v, rows_v, out_v,
             sem0, sem1, sem2, sem3):
    sems = (sem0, sem1, sem2, sem3)
    wid = _wid()

    pltpu.sync_copy(ids_hbm.at[pl.ds(wid * BPW * L, BPW * L)], idx_v)

    def start(g, u):
        dr, off, sz = _chunk(u)
        row = 2 * g + dr
        pltpu.async_copy(
            table_hbm.at[idx_v.at[pl.ds(row * L + off, sz)]],
            rows_v.at[u, pl.ds(0, sz)],
            sems[u])

    def wait(u):
        _, _, sz = _chunk(u)
        pltpu.make_async_copy(
            table_hbm.at[pl.ds(0, sz)], rows_v.at[u, pl.ds(0, sz)],
            sems[u]).wait()

    for u in range(NBUF):
        start(0, u)

    zeros = (jnp.zeros((LANES,), jnp.float32),) * (2 * NG)

    def accum(u, acc):
        _, _, sz = _chunk(u)
        rv = rows_v.at[u]

        def jbody(j, a):
            for r in range(2):
                a = list(a)
                for k in range(NG):
                    v = plsc.bitcast(
                        rv[2 * j + r, pl.ds(k * 2 * LANES, 2 * LANES)],
                        jnp.int32)
                    a[2 * k] = a[2 * k] + lax.bitcast_convert_type(
                        v << 16, jnp.float32)
                    a[2 * k + 1] = a[2 * k + 1] + lax.bitcast_convert_type(
                        v & MASK_HI, jnp.float32)
                a = tuple(a)
            return a

        return lax.fori_loop(0, sz // 2, jbody, acc)

    def gbody(g, carry):
        acc = zeros
        for u in range(NBUF):
            wait(u)

            @pl.when(g < GROUPS - 1)
            def _():
                start(g + 1, u)

            if u % 2 == 0:
                acc = accum(u, zeros)
            else:
                acc = accum(u, acc)
                row = 2 * g + u // 2
                for k in range(NG):
                    out_v[row, pl.ds(32 * k, LANES)] = acc[2 * k]
                    out_v[row, pl.ds(32 * k + 16, LANES)] = acc[2 * k + 1]
        return carry

    lax.fori_loop(0, GROUPS, gbody, 0)
    pltpu.sync_copy(out_v, out_hbm.at[pl.ds(wid * BPW, BPW)])


def kernel(input_ids, table):
    ids_flat = input_ids.reshape(B * L).astype(jnp.int32)

    convert = pl.kernel(
        _cvt_body,
        mesh=plsc.VectorSubcoreMesh(**_MESH),
        compiler_params=_PARAMS,
        out_type=jax.ShapeDtypeStruct((V, D), jnp.bfloat16),
        scratch_types=[
            pltpu.VMEM((2, RPC, D), jnp.float32),
            pltpu.VMEM((2, RPC, D), jnp.bfloat16),
            pltpu.SemaphoreType.DMA,
            pltpu.SemaphoreType.DMA,
            pltpu.SemaphoreType.DMA,
            pltpu.SemaphoreType.DMA,
        ],
    )
    tb = convert(table)

    pool = pl.kernel(
        _sc_body,
        mesh=plsc.VectorSubcoreMesh(**_MESH),
        compiler_params=_PARAMS,
        out_type=jax.ShapeDtypeStruct((B, D), jnp.float32),
        scratch_types=[
            pltpu.VMEM((BPW * L,), jnp.int32),
            pltpu.VMEM((NBUF, CH0, D), jnp.bfloat16),
            pltpu.VMEM((BPW, D), jnp.float32),
            pltpu.SemaphoreType.DMA,
            pltpu.SemaphoreType.DMA,
            pltpu.SemaphoreType.DMA,
            pltpu.SemaphoreType.DMA,
        ],
    )
    return pool(ids_flat, tb)


# speedup vs baseline: 1.0487x; 1.0487x over previous
import jax
import jax.numpy as jnp
from jax import lax
from jax.experimental import pallas as pl
from jax.experimental.pallas import tpu as pltpu
from jax.experimental.pallas import tpu_sc as plsc

V = 100000
D = 128
B = 4096
L = 200
LANES = 16
NG = D // 32

CH0 = 128
CH1 = L - CH0
NBUF = 4

NC = 2
NS = 16
NW = NC * NS
BPW = B // NW
GROUPS = BPW // 2

RPW = V // NW
RPC = 125
NCHC = RPW // RPC

MASK_HI = -65536

_PARAMS = pltpu.CompilerParams(use_tc_tiling_on_sc=False,
                               needs_layout_passes=False)
_MESH = dict(core_axis_name="c", subcore_axis_name="s")


def _wid():
    return lax.axis_index("s") * NC + lax.axis_index("c")


def _cvt_body(table_hbm, tb_hbm, in_v, out_v, semi0, semi1, semo0, semo1):
    semi = (semi0, semi1)
    semo = (semo0, semo1)
    base = _wid() * RPW

    def start_in(c, u):
        pltpu.async_copy(table_hbm.at[pl.ds(base + c * RPC, RPC)],
                         in_v.at[u], semi[u])

    def wait_in(u):
        pltpu.make_async_copy(table_hbm.at[pl.ds(0, RPC)], in_v.at[u],
                              semi[u]).wait()

    def start_out(c, u):
        pltpu.async_copy(out_v.at[u], tb_hbm.at[pl.ds(base + c * RPC, RPC)],
                         semo[u])

    def wait_out(u):
        pltpu.make_async_copy(out_v.at[u], tb_hbm.at[pl.ds(0, RPC)],
                              semo[u]).wait()

    start_in(0, 0)
    start_in(1, 1)
    for c in range(NCHC):
        u = c % 2
        wait_in(u)
        if c >= 2:
            wait_out(u)

        def rbody(j, carry, u=u):
            for r5 in range(5):
                r = 5 * j + r5
                row_in = in_v.at[u, r]
                for m in range(NG):
                    a = plsc.bitcast(row_in[pl.ds(32 * m, LANES)], jnp.int32)
                    b = plsc.bitcast(row_in[pl.ds(32 * m + 16, LANES)],
                                     jnp.int32)
                    lo = lax.shift_right_logical(a, 16)
                    hi = b & MASK_HI
                    out_v[u, r, pl.ds(32 * m, 32)] = plsc.bitcast(
                        lo | hi, jnp.bfloat16)
            return carry

        lax.fori_loop(0, RPC // 5, rbody, 0)
        start_out(c, u)
        if c + 2 < NCHC:
            start_in(c + 2, u)
    wait_out(NCHC % 2)
    wait_out((NCHC + 1) % 2)


def _chunk(u):
    return u // 2, (u % 2) * CH0, CH0 if u % 2 == 0 else CH1


def _sc_body(ids_hbm, table_hbm, out_hbm, idx_v, rows_v, out_v,
             sem0, sem1, sem2, sem3):
    sems = (sem0, sem1, sem2, sem3)
    wid = _wid()

    pltpu.sync_copy(ids_hbm.at[pl.ds(wid * BPW * L, BPW * L)], idx_v)

    def start(g, u):
        dr, off, sz = _chunk(u)
        row = 2 * g + dr
        pltpu.async_copy(
            table_hbm.at[idx_v.at[pl.ds(row * L + off, sz)]],
            rows_v.at[u, pl.ds(0, sz)],
            sems[u])

    def wait(u):
        _, _, sz = _chunk(u)
        pltpu.make_async_copy(
            table_hbm.at[pl.ds(0, sz)], rows_v.at[u, pl.ds(0, sz)],
            sems[u]).wait()

    for u in range(NBUF):
        start(0, u)

    zeros = (jnp.zeros((LANES,), jnp.float32),) * (2 * NG)

    def accum(u, acc):
        _, _, sz = _chunk(u)
        rv = rows_v.at[u]

        def jbody(j, a):
            for r in range(2):
                a = list(a)
                for k in range(NG):
                    v = plsc.bitcast(
                        rv[2 * j + r, pl.ds(k * 2 * LANES, 2 * LANES)],
                        jnp.int32)
                    a[2 * k] = a[2 * k] + lax.bitcast_convert_type(
                        v << 16, jnp.float32)
                    a[2 * k + 1] = a[2 * k + 1] + lax.bitcast_convert_type(
                        v & MASK_HI, jnp.float32)
                a = tuple(a)
            return a

        return lax.fori_loop(0, sz // 2, jbody, acc)

    def gbody(g, carry):
        acc = zeros
        for u in range(NBUF):
            wait(u)

            @pl.when(g < GROUPS - 1)
            def _():
                start(g + 1, u)

            if u % 2 == 0:
                acc = accum(u, zeros)
            else:
                acc = accum(u, acc)
                row = 2 * g + u // 2
                for k in range(NG):
                    out_v[row, pl.ds(32 * k, LANES)] = acc[2 * k]
                    out_v[row, pl.ds(32 * k + 16, LANES)] = acc[2 * k + 1]
        return carry

    lax.fori_loop(0, GROUPS, gbody, 0)
    pltpu.sync_copy(out_v, out_hbm.at[pl.ds(wid * BPW, BPW)])


def kernel(input_ids, table):
    ids_flat = input_ids.reshape(B * L).astype(jnp.int32)

    convert = pl.kernel(
        _cvt_body,
        mesh=plsc.VectorSubcoreMesh(**_MESH),
        compiler_params=_PARAMS,
        out_type=jax.ShapeDtypeStruct((V, D), jnp.bfloat16),
        scratch_types=[
            pltpu.VMEM((2, RPC, D), jnp.float32),
            pltpu.VMEM((2, RPC, D), jnp.bfloat16),
            pltpu.SemaphoreType.DMA,
            pltpu.SemaphoreType.DMA,
            pltpu.SemaphoreType.DMA,
            pltpu.SemaphoreType.DMA,
        ],
    )
    tb = convert(table)

    pool = pl.kernel(
        _sc_body,
        mesh=plsc.VectorSubcoreMesh(**_MESH),
        compiler_params=_PARAMS,
        out_type=jax.ShapeDtypeStruct((B, D), jnp.float32),
        scratch_types=[
            pltpu.VMEM((BPW * L,), jnp.int32),
            pltpu.VMEM((NBUF, CH0, D), jnp.bfloat16),
            pltpu.VMEM((BPW, D), jnp.float32),
            pltpu.SemaphoreType.DMA,
            pltpu.SemaphoreType.DMA,
            pltpu.SemaphoreType.DMA,
            pltpu.SemaphoreType.DMA,
        ],
    )
    return pool(ids_flat, tb)


# speedup vs baseline: 1.1629x; 1.1090x over previous
import jax
import jax.numpy as jnp
from jax import lax
from jax.experimental import pallas as pl
from jax.experimental.pallas import tpu as pltpu
from jax.experimental.pallas import tpu_sc as plsc

D = 128
B = 4096
L = 200
LANES = 16
NVEC = D // LANES

CH0 = 128
CH1 = L - CH0
NBUF = 4

NC = 2
NS = 16
NW = NC * NS
BPW = B // NW
GROUPS = BPW // 2


def _chunk(u):
    return u // 2, (u % 2) * CH0, CH0 if u % 2 == 0 else CH1


def _sc_body(ids_hbm, table_hbm, out_hbm, idx_v, rows_v, out_v,
             sem0, sem1, sem2, sem3):
    sems = (sem0, sem1, sem2, sem3)
    wid = lax.axis_index("s") * NC + lax.axis_index("c")

    pltpu.sync_copy(ids_hbm.at[pl.ds(wid * BPW * L, BPW * L)], idx_v)

    def start(g, u):
        dr, off, sz = _chunk(u)
        row = 2 * g + dr
        pltpu.async_copy(
            table_hbm.at[idx_v.at[pl.ds(row * L + off, sz)]],
            rows_v.at[u, pl.ds(0, sz)],
            sems[u])

    def wait(u):
        _, _, sz = _chunk(u)
        pltpu.make_async_copy(
            table_hbm.at[pl.ds(0, sz)], rows_v.at[u, pl.ds(0, sz)],
            sems[u]).wait()

    for u in range(NBUF):
        start(0, u)

    zeros = (jnp.zeros((LANES,), jnp.float32),) * NVEC

    def accum(u, acc):
        _, _, sz = _chunk(u)
        rv = rows_v.at[u]

        def jbody(j, a):
            for r in range(4):
                a = tuple(a[k] + rv[4 * j + r, pl.ds(k * LANES, LANES)]
                          for k in range(NVEC))
            return a

        return lax.fori_loop(0, sz // 4, jbody, acc)

    def gbody(g, carry):
        acc = zeros
        for u in range(NBUF):
            wait(u)

            @pl.when(g < GROUPS - 1)
            def _():
                start(g + 1, u)

            if u % 2 == 0:
                acc = accum(u, zeros)
            else:
                acc = accum(u, acc)
                row = 2 * g + u // 2
                for k in range(NVEC):
                    out_v[row, pl.ds(k * LANES, LANES)] = acc[k]
        return carry

    lax.fori_loop(0, GROUPS, gbody, 0)
    pltpu.sync_copy(out_v, out_hbm.at[pl.ds(wid * BPW, BPW)])


def kernel(input_ids, table):
    ids_flat = input_ids.reshape(B * L).astype(jnp.int32)
    f = pl.kernel(
        _sc_body,
        mesh=plsc.VectorSubcoreMesh(core_axis_name="c", subcore_axis_name="s"),
        out_type=jax.ShapeDtypeStruct((B, D), jnp.float32),
        scratch_types=[
            pltpu.VMEM((BPW * L,), jnp.int32),
            pltpu.VMEM((NBUF, CH0, D), jnp.float32),
            pltpu.VMEM((BPW, D), jnp.float32),
            pltpu.SemaphoreType.DMA,
            pltpu.SemaphoreType.DMA,
            pltpu.SemaphoreType.DMA,
            pltpu.SemaphoreType.DMA,
        ],
    )
    return f(ids_flat, table)
